# baseline (speedup 1.0000x reference)
import functools

import jax
import jax.numpy as jnp
from jax import lax
from jax.experimental import pallas as pl
from jax.experimental.pallas import tpu as pltpu
from jax.experimental.pallas import tpu_sc as plsc

VOCAB = 1000000
EMBED_DIM = 64
PADDED_DIM = 128
BATCH = 4096
HIST = 200

NUM_CORES = 2
NUM_SUBCORES = 16
NUM_WORKERS = NUM_CORES * NUM_SUBCORES
LANES = 16

BCHUNK = BATCH // NUM_WORKERS
NBUF = 4
DEPTH = NBUF - 1
RPITCH = 131
XPITCH = 129


def _gather_kernel(x_hbm, table_hbm, out_hbm, x_v, xT_v,
                   rv0, rv1, rv2, rv3, ob0, ob1, gsems, osems):
    wid = lax.axis_index("s") * NUM_CORES + lax.axis_index("c")
    wcol = wid * BCHUNK
    lane_iota = lax.iota(jnp.int32, LANES)
    rvs = (rv0, rv1, rv2, rv3)
    obs = (ob0, ob1)

    for half in range(2):
        pltpu.sync_copy(x_hbm.at[pl.ds(wcol + 64 * half, 64)], x_v)

        def xt_body(b, carry, half=half):
            col = jnp.broadcast_to(64 * half + b, (LANES,)).astype(jnp.int32)
            for hc in range(13):
                h0 = min(hc * LANES, HIST - LANES)
                v = x_v[b, pl.ds(h0, LANES)]
                plsc.store_scatter(xT_v, [h0 + lane_iota, col], v)
            return carry
        lax.fori_loop(0, 64, xt_body, 0)

    def gather_copy(h, b):
        src = table_hbm.at[xT_v.at[h, pl.ds(0, PADDED_DIM)]]
        return pltpu.make_async_copy(src, rvs[b], gsems[b])

    def out_copy(h, ob):
        dst = out_hbm.at[h, :, pl.ds(wcol, BCHUNK)]
        src = obs[ob].at[:, pl.ds(0, BCHUNK)]
        return pltpu.make_async_copy(src, dst, osems[ob])

    def transpose_rows(b, ob):
        def tb(g, carry):
            for k in range(8):
                bb = g * 8 + k
                col = jnp.broadcast_to(bb, (LANES,)).astype(jnp.int32)
                for fc in range(EMBED_DIM // LANES):
                    v = rvs[b][bb, pl.ds(fc * LANES, LANES)]
                    plsc.store_scatter(
                        obs[ob], [fc * LANES + lane_iota, col], v)
            return carry
        lax.fori_loop(0, BCHUNK // 8, tb, 0)

    def visit(h, b):
        ob = b % 2
        gather_copy(h, b).wait()

        @pl.when(h + DEPTH < HIST)
        def _():
            gather_copy(h + DEPTH, (b + DEPTH) % NBUF).start()

        @pl.when(h >= 2)
        def _():
            out_copy(h - 2, ob).wait()
        transpose_rows(b, ob)
        out_copy(h, ob).start()

    for h in range(DEPTH):
        gather_copy(h, h).start()

    def group(p, carry):
        for b in range(NBUF):
            visit(NBUF * p + b, b)
        return carry

    lax.fori_loop(0, HIST // NBUF, group, 0)

    out_copy(HIST - 2, 0).wait()
    out_copy(HIST - 1, 1).wait()


@jax.jit
def _embed(x, table):
    table_p = jnp.pad(table, ((0, 0), (0, PADDED_DIM - EMBED_DIM)))
    mesh = plsc.VectorSubcoreMesh(
        core_axis_name="c", subcore_axis_name="s",
        num_cores=NUM_CORES, num_subcores=NUM_SUBCORES,
    )
    run = functools.partial(
        pl.kernel,
        out_type=jax.ShapeDtypeStruct((HIST, EMBED_DIM, BATCH), jnp.float32),
        mesh=mesh,
        scratch_types=[
            pltpu.VMEM((64, HIST), jnp.int32),
            pltpu.VMEM((HIST, XPITCH), jnp.int32),
            pltpu.VMEM((BCHUNK, PADDED_DIM), jnp.float32),
            pltpu.VMEM((BCHUNK, PADDED_DIM), jnp.float32),
            pltpu.VMEM((BCHUNK, PADDED_DIM), jnp.float32),
            pltpu.VMEM((BCHUNK, PADDED_DIM), jnp.float32),
            pltpu.VMEM((EMBED_DIM, RPITCH), jnp.float32),
            pltpu.VMEM((EMBED_DIM, RPITCH), jnp.float32),
            [pltpu.SemaphoreType.DMA] * NBUF,
            [pltpu.SemaphoreType.DMA] * 2,
        ],
        compiler_params=pltpu.CompilerParams(
            use_tc_tiling_on_sc=False, needs_layout_passes=False),
    )(_gather_kernel)
    out_t = run(x, table_p)
    return jnp.transpose(out_t, (2, 0, 1))


def kernel(x, table):
    return _embed(x, table)


# speedup vs baseline: 1.2570x; 1.2570x over previous
import functools

import jax
import jax.numpy as jnp
from jax import lax
from jax.experimental import pallas as pl
from jax.experimental.pallas import tpu as pltpu
from jax.experimental.pallas import tpu_sc as plsc

VOCAB = 1000000
EMBED_DIM = 64
PADDED_DIM = 128
BATCH = 4096
HIST = 200

NUM_CORES = 2
NUM_SUBCORES = 16
NUM_WORKERS = NUM_CORES * NUM_SUBCORES

XROWS = BATCH // NUM_WORKERS
NUM_CHUNKS = XROWS
NBUF = 4
DEPTH = NBUF - 1


def _gather_kernel(x_hbm, table_hbm, out_hbm, idx_all, rows_v, gsems, ssems):
    wid = lax.axis_index("s") * NUM_CORES + lax.axis_index("c")
    wrow = wid * XROWS

    pltpu.sync_copy(x_hbm.at[pl.ds(wrow, XROWS)], idx_all)

    def gather_copy(c, b):
        src = table_hbm.at[idx_all.at[c]]
        return pltpu.make_async_copy(src, rows_v.at[b], gsems[b])

    def store_copy(c, b):
        dst = out_hbm.at[wrow + c, :, pl.ds(0, EMBED_DIM)]
        src = rows_v.at[b, :, pl.ds(0, EMBED_DIM)]
        return pltpu.make_async_copy(src, dst, ssems[b])

    def visit(c, b):
        gather_copy(c, b).wait()
        store_copy(c, b).start()
        h = c + DEPTH
        hb = (b + DEPTH) % NBUF

        @pl.when(h < NUM_CHUNKS)
        def _():
            @pl.when(h >= NBUF)
            def _():
                store_copy(h - NBUF, hb).wait()
            gather_copy(h, hb).start()

    for h in range(DEPTH):
        gather_copy(h, h).start()

    def group(p, carry):
        for b in range(NBUF):
            visit(NBUF * p + b, b)
        return carry

    lax.fori_loop(0, NUM_CHUNKS // NBUF, group, 0)

    for b in range(NBUF):
        store_copy(NUM_CHUNKS - NBUF + b, b).wait()


@jax.jit
def _embed(x, table):
    table_p = jnp.pad(table, ((0, 0), (0, PADDED_DIM - EMBED_DIM)))
    mesh = plsc.VectorSubcoreMesh(
        core_axis_name="c", subcore_axis_name="s",
        num_cores=NUM_CORES, num_subcores=NUM_SUBCORES,
    )
    run = functools.partial(
        pl.kernel,
        out_type=jax.ShapeDtypeStruct((BATCH, HIST, PADDED_DIM), jnp.float32),
        mesh=mesh,
        scratch_types=[
            pltpu.VMEM((XROWS, HIST), jnp.int32),
            pltpu.VMEM((NBUF, HIST, PADDED_DIM), jnp.float32),
            [pltpu.SemaphoreType.DMA] * NBUF,
            [pltpu.SemaphoreType.DMA] * NBUF,
        ],
        compiler_params=pltpu.CompilerParams(use_tc_tiling_on_sc=False),
    )(_gather_kernel)
    out_p = run(x, table_p)
    return out_p[:, :, :EMBED_DIM]


def kernel(x, table):
    return _embed(x, table)
